# baseline (speedup 1.0000x reference)
import functools

import jax
import jax.numpy as jnp
import numpy as np
from jax import lax
from jax.experimental import pallas as pl
from jax.experimental.pallas import tpu as pltpu
from jax.experimental.pallas import tpu_sc as plsc

_OFFS = ((-1, 1), (0, 1), (1, 1), (1, 0), (1, -1), (0, -1), (-1, -1), (-1, 0))
_H = 512
_W = 512
_T = _H * _W
_N = 8

_NTEC = 32
_SLICES = _NTEC // _N
_SLICE = _T // _SLICES
_UNROLL = 16
_NCOPY = 4


def _lbp_body(img_ref, out_ref, code_ref):
    x = img_ref[0]
    gray = 0.299 * x[0] + 0.587 * x[1] + 0.114 * x[2]
    g = gray.astype(jnp.bfloat16)
    pg = jnp.pad(g, ((1, 1), (1, 1)))
    xv = [jax.lax.slice(pg, (0, dx), (_H + 2, dx + _W)) for dx in range(3)]
    code = jnp.zeros((_H, _W), jnp.bfloat16)
    for k, (dy, dx) in enumerate(_OFFS):
        nb = jax.lax.slice(xv[dx + 1], (1 + dy, 0), (1 + dy + _H, _W))
        code = code + jnp.where(nb - g >= 0.0, jnp.bfloat16(2 ** k),
                                jnp.bfloat16(0.0))

    code = code.astype(jnp.float32)
    code_ref[0] = code.reshape(_T // 128, 128)

    s1 = jnp.sum(code)
    s2 = jnp.sum(code * code)
    mean2 = s1 / np.float32(_T)
    var2 = (s2 - mean2 * s1) / np.float32(_T - 1)
    out_ref[0, 0] = (code - mean2) * jax.lax.rsqrt(var2)


def _sc_hist_body(codes_hbm, out_hbm, buf, hist, acc):
    c = lax.axis_index("c")
    s = lax.axis_index("s")
    wid = s * 2 + c
    n = wid % _N
    sl = wid // _N
    base = n * _T + sl * _SLICE

    pltpu.sync_copy(codes_hbm.at[pl.ds(base, _SLICE)], buf)

    zeros = jnp.zeros((16,), jnp.float32)
    for j in range(256 * _NCOPY):
        hist[pl.ds(j * 16, 16)] = zeros

    lanes = lax.iota(jnp.int32, 16)
    ones = jnp.ones((16,), jnp.float32)

    def inner(i, carry):
        for u in range(_UNROLL):
            v = buf[pl.ds((i * _UNROLL + u) * 16, 16)].astype(jnp.int32)
            idx = v * 16 + lanes + (u % _NCOPY) * 4096
            plsc.addupdate_scatter(hist, [idx], ones)
        return carry

    lax.fori_loop(0, _SLICE // (16 * _UNROLL), inner, 0)

    for j in range(256):
        a = hist[pl.ds(j * 16, 16)]
        for cpy in range(1, _NCOPY):
            a = a + hist[pl.ds(cpy * 4096 + j * 16, 16)]
        hist[pl.ds(j * 16, 16)] = a
    for j in range(16):
        a = jnp.zeros((16,), jnp.float32)
        for b in range(16):
            s = jnp.sum(hist[pl.ds((j * 16 + b) * 16, 16)])
            a = jnp.where(lanes == b, s, a)
        acc[pl.ds(j * 16, 16)] = a

    pltpu.sync_copy(acc, out_hbm.at[wid])


def _sc_hist(codes_flat):
    return pl.kernel(
        _sc_hist_body,
        out_type=jax.ShapeDtypeStruct((_NTEC, 256), jnp.float32),
        mesh=plsc.VectorSubcoreMesh(core_axis_name="c", subcore_axis_name="s",
                                    num_cores=2, num_subcores=16),
        scratch_types=[
            pltpu.VMEM((_SLICE,), jnp.float32),
            pltpu.VMEM((16 * 256 * _NCOPY,), jnp.float32),
            pltpu.VMEM((256,), jnp.float32),
        ],
        compiler_params=pltpu.CompilerParams(needs_layout_passes=False),
    )(codes_flat)


def _hist_norm_body(part_ref, hist_ref):
    h = part_ref[...].reshape(_SLICES, _N, 256).sum(axis=0)
    hmean = jnp.sum(h, axis=1, keepdims=True) / np.float32(256.0)
    hvar = jnp.sum((h - hmean) ** 2, axis=1, keepdims=True) / np.float32(255.0)
    hist_ref[...] = (h - hmean) * jax.lax.rsqrt(hvar)


def kernel(img):
    n = img.shape[0]
    out, codes = pl.pallas_call(
        _lbp_body,
        grid=(n,),
        in_specs=[pl.BlockSpec((1, 3, _H, _W), lambda i: (i, 0, 0, 0))],
        out_specs=[
            pl.BlockSpec((1, 1, _H, _W), lambda i: (i, 0, 0, 0)),
            pl.BlockSpec((1, _T // 128, 128), lambda i: (i, 0, 0)),
        ],
        out_shape=[
            jax.ShapeDtypeStruct((n, 1, _H, _W), jnp.float32),
            jax.ShapeDtypeStruct((n, _T // 128, 128), jnp.float32),
        ],
    )(img)

    parts = _sc_hist(codes.reshape(n * _T))

    hist = pl.pallas_call(
        _hist_norm_body,
        in_specs=[pl.BlockSpec((_NTEC, 256), lambda: (0, 0))],
        out_specs=pl.BlockSpec((_N, 256), lambda: (0, 0)),
        out_shape=jax.ShapeDtypeStruct((_N, 256), jnp.float32),
    )(parts)

    return hist, out


# speedup vs baseline: 1.6458x; 1.6458x over previous
import functools

import jax
import jax.numpy as jnp
import numpy as np
from jax import lax
from jax.experimental import pallas as pl
from jax.experimental.pallas import tpu as pltpu
from jax.experimental.pallas import tpu_sc as plsc

_OFFS = ((-1, 1), (0, 1), (1, 1), (1, 0), (1, -1), (0, -1), (-1, -1), (-1, 0))
_H = 512
_W = 512
_T = _H * _W
_N = 8

_NTEC = 32
_SLICES = _NTEC // _N
_SLICE = _T // _SLICES
_UNROLL = 16


def _lbp_body(img_ref, out_ref, code_ref):
    x = img_ref[0]
    gray = 0.299 * x[0] + 0.587 * x[1] + 0.114 * x[2]
    g = gray.astype(jnp.bfloat16)
    pg = jnp.pad(g, ((1, 1), (1, 1)))
    xv = [jax.lax.slice(pg, (0, dx), (_H + 2, dx + _W)) for dx in range(3)]
    code = jnp.zeros((_H, _W), jnp.bfloat16)
    for k, (dy, dx) in enumerate(_OFFS):
        nb = jax.lax.slice(xv[dx + 1], (1 + dy, 0), (1 + dy + _H, _W))
        code = code + jnp.where(nb - g >= 0.0, jnp.bfloat16(2 ** k),
                                jnp.bfloat16(0.0))

    code = code.astype(jnp.float32)
    code_ref[0] = code.reshape(_T // 128, 128)

    s1 = jnp.sum(code)
    s2 = jnp.sum(code * code)
    mean2 = s1 / np.float32(_T)
    var2 = (s2 - mean2 * s1) / np.float32(_T - 1)
    out_ref[0, 0] = (code - mean2) * jax.lax.rsqrt(var2)


def _sc_hist_body(codes_hbm, out_hbm, buf, hist, acc):
    c = lax.axis_index("c")
    s = lax.axis_index("s")
    wid = s * 2 + c
    n = wid % _N
    sl = wid // _N
    base = n * _T + sl * _SLICE

    pltpu.sync_copy(codes_hbm.at[pl.ds(base, _SLICE)], buf)

    zeros = jnp.zeros((16,), jnp.float32)
    lanes = lax.iota(jnp.int32, 16)
    ones = jnp.ones((16,), jnp.float32)

    @plsc.parallel_loop(0, 256, 1, unroll=4)
    def _zero(j):
        hist[pl.ds(j * 16, 16)] = zeros

    @plsc.parallel_loop(0, _SLICE // 16, 1, unroll=_UNROLL)
    def _scatter(i):
        v = buf[pl.ds(i * 16, 16)].astype(jnp.int32)
        idx = v * 16 + lanes
        plsc.addupdate_scatter(hist, [idx], ones)

    @plsc.parallel_loop(0, 16, 1, unroll=2)
    def _reduce(j):
        a = jnp.zeros((16,), jnp.float32)
        for b in range(16):
            s = jnp.sum(hist[pl.ds((j * 16 + b) * 16, 16)])
            a = jnp.where(lanes == b, s, a)
        acc[pl.ds(j * 16, 16)] = a

    pltpu.sync_copy(acc, out_hbm.at[wid])


def _sc_hist(codes_flat):
    return pl.kernel(
        _sc_hist_body,
        out_type=jax.ShapeDtypeStruct((_NTEC, 256), jnp.float32),
        mesh=plsc.VectorSubcoreMesh(core_axis_name="c", subcore_axis_name="s",
                                    num_cores=2, num_subcores=16),
        scratch_types=[
            pltpu.VMEM((_SLICE,), jnp.float32),
            pltpu.VMEM((16 * 256,), jnp.float32),
            pltpu.VMEM((256,), jnp.float32),
        ],
        compiler_params=pltpu.CompilerParams(needs_layout_passes=False),
    )(codes_flat)


def _hist_norm_body(part_ref, hist_ref):
    h = part_ref[...].reshape(_SLICES, _N, 256).sum(axis=0)
    hmean = jnp.sum(h, axis=1, keepdims=True) / np.float32(256.0)
    hvar = jnp.sum((h - hmean) ** 2, axis=1, keepdims=True) / np.float32(255.0)
    hist_ref[...] = (h - hmean) * jax.lax.rsqrt(hvar)


def kernel(img):
    n = img.shape[0]
    out, codes = pl.pallas_call(
        _lbp_body,
        grid=(n,),
        in_specs=[pl.BlockSpec((1, 3, _H, _W), lambda i: (i, 0, 0, 0))],
        out_specs=[
            pl.BlockSpec((1, 1, _H, _W), lambda i: (i, 0, 0, 0)),
            pl.BlockSpec((1, _T // 128, 128), lambda i: (i, 0, 0)),
        ],
        out_shape=[
            jax.ShapeDtypeStruct((n, 1, _H, _W), jnp.float32),
            jax.ShapeDtypeStruct((n, _T // 128, 128), jnp.float32),
        ],
    )(img)

    parts = _sc_hist(codes.reshape(n * _T))

    hist = pl.pallas_call(
        _hist_norm_body,
        in_specs=[pl.BlockSpec((_NTEC, 256), lambda: (0, 0))],
        out_specs=pl.BlockSpec((_N, 256), lambda: (0, 0)),
        out_shape=jax.ShapeDtypeStruct((_N, 256), jnp.float32),
    )(parts)

    return hist, out


# speedup vs baseline: 1.7206x; 1.0455x over previous
import functools

import jax
import jax.numpy as jnp
import numpy as np
from jax import lax
from jax.experimental import pallas as pl
from jax.experimental.pallas import tpu as pltpu
from jax.experimental.pallas import tpu_sc as plsc

_OFFS = ((-1, 1), (0, 1), (1, 1), (1, 0), (1, -1), (0, -1), (-1, -1), (-1, 0))
_H = 512
_W = 512
_T = _H * _W
_N = 8

_NTEC = 32
_SLICES = _NTEC // _N
_SLICE = _T // _SLICES
_UNROLL = 16


def _lbp_body(img_ref, out_ref, code_ref):
    x = img_ref[0]
    gray = 0.299 * x[0] + 0.587 * x[1] + 0.114 * x[2]
    g = gray.astype(jnp.bfloat16)
    xv = [
        jnp.pad(g, ((1, 1), (1, 0)))[:, :_W],
        jnp.pad(g, ((1, 1), (0, 0))),
        jnp.pad(g, ((1, 1), (0, 1)))[:, 1:],
    ]
    code = jnp.zeros((_H, _W), jnp.bfloat16)
    for k, (dy, dx) in enumerate(_OFFS):
        nb = jax.lax.slice(xv[dx + 1], (1 + dy, 0), (1 + dy + _H, _W))
        code = code + jnp.where(nb - g >= 0.0, jnp.bfloat16(2 ** k),
                                jnp.bfloat16(0.0))

    code = code.astype(jnp.float32)
    code_ref[0] = code.reshape(_T // 128, 128)

    s1 = jnp.sum(code)
    s2 = jnp.sum(code * code)
    mean2 = s1 / np.float32(_T)
    var2 = (s2 - mean2 * s1) / np.float32(_T - 1)
    out_ref[0, 0] = (code - mean2) * jax.lax.rsqrt(var2)


def _sc_hist_body(codes_hbm, out_hbm, buf, hist, acc):
    c = lax.axis_index("c")
    s = lax.axis_index("s")
    wid = s * 2 + c
    n = wid % _N
    sl = wid // _N
    base = n * _T + sl * _SLICE

    pltpu.sync_copy(codes_hbm.at[pl.ds(base, _SLICE)], buf)

    zeros = jnp.zeros((16,), jnp.float32)
    lanes = lax.iota(jnp.int32, 16)
    ones = jnp.ones((16,), jnp.float32)

    @plsc.parallel_loop(0, 256, 1, unroll=4)
    def _zero(j):
        hist[pl.ds(j * 16, 16)] = zeros

    @plsc.parallel_loop(0, _SLICE // 16, 1, unroll=_UNROLL)
    def _scatter(i):
        v = buf[pl.ds(i * 16, 16)].astype(jnp.int32)
        idx = v * 16 + lanes
        plsc.addupdate_scatter(hist, [idx], ones)

    @plsc.parallel_loop(0, 16, 1, unroll=2)
    def _reduce(j):
        a = jnp.zeros((16,), jnp.float32)
        for b in range(16):
            s = jnp.sum(hist[pl.ds((j * 16 + b) * 16, 16)])
            a = jnp.where(lanes == b, s, a)
        acc[pl.ds(j * 16, 16)] = a

    pltpu.sync_copy(acc, out_hbm.at[wid])


def _sc_hist(codes_flat):
    return pl.kernel(
        _sc_hist_body,
        out_type=jax.ShapeDtypeStruct((_NTEC, 256), jnp.float32),
        mesh=plsc.VectorSubcoreMesh(core_axis_name="c", subcore_axis_name="s",
                                    num_cores=2, num_subcores=16),
        scratch_types=[
            pltpu.VMEM((_SLICE,), jnp.float32),
            pltpu.VMEM((16 * 256,), jnp.float32),
            pltpu.VMEM((256,), jnp.float32),
        ],
        compiler_params=pltpu.CompilerParams(needs_layout_passes=False),
    )(codes_flat)


def _hist_norm_body(part_ref, hist_ref):
    h = part_ref[...].reshape(_SLICES, _N, 256).sum(axis=0)
    hmean = jnp.sum(h, axis=1, keepdims=True) / np.float32(256.0)
    hvar = jnp.sum((h - hmean) ** 2, axis=1, keepdims=True) / np.float32(255.0)
    hist_ref[...] = (h - hmean) * jax.lax.rsqrt(hvar)


def kernel(img):
    n = img.shape[0]
    out, codes = pl.pallas_call(
        _lbp_body,
        grid=(n,),
        in_specs=[pl.BlockSpec((1, 3, _H, _W), lambda i: (i, 0, 0, 0))],
        out_specs=[
            pl.BlockSpec((1, 1, _H, _W), lambda i: (i, 0, 0, 0)),
            pl.BlockSpec((1, _T // 128, 128), lambda i: (i, 0, 0)),
        ],
        out_shape=[
            jax.ShapeDtypeStruct((n, 1, _H, _W), jnp.float32),
            jax.ShapeDtypeStruct((n, _T // 128, 128), jnp.float32),
        ],
    )(img)

    parts = _sc_hist(codes.reshape(n * _T))

    hist = pl.pallas_call(
        _hist_norm_body,
        in_specs=[pl.BlockSpec((_NTEC, 256), lambda: (0, 0))],
        out_specs=pl.BlockSpec((_N, 256), lambda: (0, 0)),
        out_shape=jax.ShapeDtypeStruct((_N, 256), jnp.float32),
    )(parts)

    return hist, out


# speedup vs baseline: 1.7729x; 1.0304x over previous
import functools

import jax
import jax.numpy as jnp
import numpy as np
from jax import lax
from jax.experimental import pallas as pl
from jax.experimental.pallas import tpu as pltpu
from jax.experimental.pallas import tpu_sc as plsc

_OFFS = ((-1, 1), (0, 1), (1, 1), (1, 0), (1, -1), (0, -1), (-1, -1), (-1, 0))
_H = 512
_W = 512
_T = _H * _W
_N = 8

_NTEC = 32
_SLICES = _NTEC // _N
_SLICE = _T // _SLICES
_UNROLL = 16
_NCHUNK = 4
_CHUNK = _SLICE // _NCHUNK


def _lbp_body(img_ref, out_ref, code_ref):
    x = img_ref[0]
    gray = 0.299 * x[0] + 0.587 * x[1] + 0.114 * x[2]
    g = gray.astype(jnp.bfloat16)
    xv = [
        jnp.pad(g, ((1, 1), (1, 0)))[:, :_W],
        jnp.pad(g, ((1, 1), (0, 0))),
        jnp.pad(g, ((1, 1), (0, 1)))[:, 1:],
    ]
    code = jnp.zeros((_H, _W), jnp.bfloat16)
    for k, (dy, dx) in enumerate(_OFFS):
        nb = jax.lax.slice(xv[dx + 1], (1 + dy, 0), (1 + dy + _H, _W))
        code = code + jnp.where(nb - g >= 0.0, jnp.bfloat16(2 ** k),
                                jnp.bfloat16(0.0))

    code = code.astype(jnp.float32)
    code_ref[0] = code.reshape(_T // 128, 128)

    s1 = jnp.sum(code)
    s2 = jnp.sum(code * code)
    mean2 = s1 / np.float32(_T)
    var2 = (s2 - mean2 * s1) / np.float32(_T - 1)
    out_ref[0, 0] = (code - mean2) * jax.lax.rsqrt(var2)


def _sc_hist_body(codes_hbm, out_hbm, buf0, buf1, hist, acc, sem0, sem1):
    c = lax.axis_index("c")
    s = lax.axis_index("s")
    wid = s * 2 + c
    n = wid % _N
    sl = wid // _N
    base = n * _T + sl * _SLICE

    bufs = (buf0, buf1)
    sems = (sem0, sem1)
    copies = [None, None]
    copies[0] = pltpu.async_copy(
        codes_hbm.at[pl.ds(base, _CHUNK)], buf0, sem0)

    zeros = jnp.zeros((16,), jnp.float32)
    lanes = lax.iota(jnp.int32, 16)
    ones = jnp.ones((16,), jnp.float32)

    @plsc.parallel_loop(0, 256, 1, unroll=4)
    def _zero(j):
        hist[pl.ds(j * 16, 16)] = zeros

    for k in range(_NCHUNK):
        if k + 1 < _NCHUNK:
            copies[(k + 1) % 2] = pltpu.async_copy(
                codes_hbm.at[pl.ds(base + (k + 1) * _CHUNK, _CHUNK)],
                bufs[(k + 1) % 2], sems[(k + 1) % 2])
        copies[k % 2].wait()
        cur = bufs[k % 2]

        @plsc.parallel_loop(0, _CHUNK // 16, 1, unroll=_UNROLL)
        def _scatter(i):
            v = cur[pl.ds(i * 16, 16)].astype(jnp.int32)
            idx = v * 16 + lanes
            plsc.addupdate_scatter(hist, [idx], ones)

    @plsc.parallel_loop(0, 16, 1, unroll=2)
    def _reduce(j):
        a = jnp.zeros((16,), jnp.float32)
        for b in range(16):
            s = jnp.sum(hist[pl.ds((j * 16 + b) * 16, 16)])
            a = jnp.where(lanes == b, s, a)
        acc[pl.ds(j * 16, 16)] = a

    pltpu.sync_copy(acc, out_hbm.at[wid])


def _sc_hist(codes_flat):
    return pl.kernel(
        _sc_hist_body,
        out_type=jax.ShapeDtypeStruct((_NTEC, 256), jnp.float32),
        mesh=plsc.VectorSubcoreMesh(core_axis_name="c", subcore_axis_name="s",
                                    num_cores=2, num_subcores=16),
        scratch_types=[
            pltpu.VMEM((_CHUNK,), jnp.float32),
            pltpu.VMEM((_CHUNK,), jnp.float32),
            pltpu.VMEM((16 * 256,), jnp.float32),
            pltpu.VMEM((256,), jnp.float32),
            pltpu.SemaphoreType.DMA,
            pltpu.SemaphoreType.DMA,
        ],
        compiler_params=pltpu.CompilerParams(needs_layout_passes=False),
    )(codes_flat)


def _hist_norm_body(part_ref, hist_ref):
    h = part_ref[...].reshape(_SLICES, _N, 256).sum(axis=0)
    hmean = jnp.sum(h, axis=1, keepdims=True) / np.float32(256.0)
    hvar = jnp.sum((h - hmean) ** 2, axis=1, keepdims=True) / np.float32(255.0)
    hist_ref[...] = (h - hmean) * jax.lax.rsqrt(hvar)


def kernel(img):
    n = img.shape[0]
    out, codes = pl.pallas_call(
        _lbp_body,
        grid=(n,),
        in_specs=[pl.BlockSpec((1, 3, _H, _W), lambda i: (i, 0, 0, 0))],
        out_specs=[
            pl.BlockSpec((1, 1, _H, _W), lambda i: (i, 0, 0, 0)),
            pl.BlockSpec((1, _T // 128, 128), lambda i: (i, 0, 0)),
        ],
        out_shape=[
            jax.ShapeDtypeStruct((n, 1, _H, _W), jnp.float32),
            jax.ShapeDtypeStruct((n, _T // 128, 128), jnp.float32),
        ],
    )(img)

    parts = _sc_hist(codes.reshape(n * _T))

    hist = pl.pallas_call(
        _hist_norm_body,
        in_specs=[pl.BlockSpec((_NTEC, 256), lambda: (0, 0))],
        out_specs=pl.BlockSpec((_N, 256), lambda: (0, 0)),
        out_shape=jax.ShapeDtypeStruct((_N, 256), jnp.float32),
    )(parts)

    return hist, out
